# baseline (device time: 26095 ns/iter reference)
import jax
import jax.numpy as jnp
from jax import lax
from jax.experimental import pallas as pl
from jax.experimental.pallas import tpu as pltpu

N_DEV = 4


def kernel(x, w_mat):
    m, _ = x.shape
    _, n = w_mat.shape
    m_blk = m // N_DEV
    nh = n // 2

    def body(x_ref, w_ref, out_ref,
             snd_feed, snd_leaf, snd_relay, rcv_feed, rcv_leaf, rcv_red,
             feed_sems, leaf_sems, red_sems,
             feed_send_sems, leaf_send_sems, red_send_sems):
        my = lax.axis_index("i")
        left = (my - 1) % N_DEV
        right = (my + 1) % N_DEV

        barrier_sem = pltpu.get_barrier_semaphore()
        for nbr in (left, right):
            pl.semaphore_signal(
                barrier_sem, inc=1,
                device_id=(nbr,), device_id_type=pl.DeviceIdType.MESH,
            )
        pl.semaphore_wait(barrier_sem, 2)

        def partial_half(c, lo):
            rows = x_ref[pl.ds(c * m_blk, m_blk), :]
            return jnp.dot(rows, w_ref[:, lo:lo + nh],
                           preferred_element_type=jnp.float32)

        def copy(src, dst, send_sem, recv_sem, dev):
            return pltpu.make_async_remote_copy(
                src_ref=src, dst_ref=dst, send_sem=send_sem, recv_sem=recv_sem,
                device_id=(dev,), device_id_type=pl.DeviceIdType.MESH,
            )

        snd_feed[0, :, :] = partial_half((my + 2) % N_DEV, 0)
        feed_a = copy(snd_feed.at[0], rcv_feed.at[0],
                      feed_send_sems.at[0], feed_sems.at[0], left)
        feed_a.start()
        snd_feed[1, :, :] = partial_half((my + 2) % N_DEV, nh)
        feed_b = copy(snd_feed.at[1], rcv_feed.at[1],
                      feed_send_sems.at[1], feed_sems.at[1], right)
        feed_b.start()

        snd_leaf[0, :, :] = partial_half((my + 1) % N_DEV, 0)
        leaf_a = copy(snd_leaf.at[0], rcv_leaf.at[0],
                      leaf_send_sems.at[0], leaf_sems.at[0], right)
        leaf_a.start()
        snd_leaf[1, :, :] = partial_half((my - 1) % N_DEV, nh)
        leaf_b = copy(snd_leaf.at[1], rcv_leaf.at[1],
                      leaf_send_sems.at[1], leaf_sems.at[1], left)
        leaf_b.start()

        p2a = partial_half((my - 1) % N_DEV, 0)
        p2b = partial_half((my + 1) % N_DEV, nh)
        loc_a = partial_half(my, 0)
        loc_b = partial_half(my, nh)

        feed_a_recv = copy(snd_feed.at[0], rcv_feed.at[0],
                           feed_send_sems.at[0], feed_sems.at[0], right)
        feed_a_recv.wait_recv()
        snd_relay[0, :, :] = rcv_feed[0, :, :] + p2a
        relay_a = copy(snd_relay.at[0], rcv_red.at[0],
                       red_send_sems.at[0], red_sems.at[0], left)
        relay_a.start()

        feed_b_recv = copy(snd_feed.at[1], rcv_feed.at[1],
                           feed_send_sems.at[1], feed_sems.at[1], left)
        feed_b_recv.wait_recv()
        snd_relay[1, :, :] = rcv_feed[1, :, :] + p2b
        relay_b = copy(snd_relay.at[1], rcv_red.at[1],
                       red_send_sems.at[1], red_sems.at[1], right)
        relay_b.start()

        leaf_a_recv = copy(snd_leaf.at[0], rcv_leaf.at[0],
                           leaf_send_sems.at[0], leaf_sems.at[0], left)
        red_a_recv = copy(snd_relay.at[0], rcv_red.at[0],
                          red_send_sems.at[0], red_sems.at[0], right)
        leaf_a_recv.wait_recv()
        red_a_recv.wait_recv()
        out_ref[:, :nh] = jnp.maximum(
            loc_a + rcv_leaf[0, :, :] + rcv_red[0, :, :], 0.0)

        leaf_b_recv = copy(snd_leaf.at[1], rcv_leaf.at[1],
                           leaf_send_sems.at[1], leaf_sems.at[1], right)
        red_b_recv = copy(snd_relay.at[1], rcv_red.at[1],
                          red_send_sems.at[1], red_sems.at[1], left)
        leaf_b_recv.wait_recv()
        red_b_recv.wait_recv()
        out_ref[:, nh:] = jnp.maximum(
            loc_b + rcv_leaf[1, :, :] + rcv_red[1, :, :], 0.0)

        for d in (feed_a, feed_b, leaf_a, leaf_b, relay_a, relay_b):
            d.wait_send()

    half = (m_blk, nh)
    return pl.pallas_call(
        body,
        out_shape=jax.ShapeDtypeStruct((m_blk, n), jnp.float32),
        in_specs=[
            pl.BlockSpec(memory_space=pltpu.VMEM),
            pl.BlockSpec(memory_space=pltpu.VMEM),
        ],
        out_specs=pl.BlockSpec(memory_space=pltpu.VMEM),
        scratch_shapes=[
            pltpu.VMEM((2,) + half, jnp.float32),
            pltpu.VMEM((2,) + half, jnp.float32),
            pltpu.VMEM((2,) + half, jnp.float32),
            pltpu.VMEM((2,) + half, jnp.float32),
            pltpu.VMEM((2,) + half, jnp.float32),
            pltpu.VMEM((2,) + half, jnp.float32),
            pltpu.SemaphoreType.DMA((2,)),
            pltpu.SemaphoreType.DMA((2,)),
            pltpu.SemaphoreType.DMA((2,)),
            pltpu.SemaphoreType.DMA((2,)),
            pltpu.SemaphoreType.DMA((2,)),
            pltpu.SemaphoreType.DMA((2,)),
        ],
        compiler_params=pltpu.CompilerParams(collective_id=0),
    )(x, w_mat)


# device time: 17710 ns/iter; 1.4735x vs baseline; 1.4735x over previous
import jax
import jax.numpy as jnp
from jax import lax
from jax.experimental import pallas as pl
from jax.experimental.pallas import tpu as pltpu

N_DEV = 4


def kernel(x, w_mat):
    m, _ = x.shape
    _, n = w_mat.shape
    m_blk = m // N_DEV
    nh = n // 2

    def body(x_ref, w_ref, out_ref,
             snd_feed, snd_leaf, snd_relay, rcv_feed, rcv_leaf, rcv_red,
             feed_sems, leaf_sems, red_sems,
             feed_send_sems, leaf_send_sems, red_send_sems):
        my = lax.axis_index("i")
        left = (my - 1) % N_DEV
        right = (my + 1) % N_DEV

        barrier_sem = pltpu.get_barrier_semaphore()
        for nbr in (left, right):
            pl.semaphore_signal(
                barrier_sem, inc=1,
                device_id=(nbr,), device_id_type=pl.DeviceIdType.MESH,
            )
        pl.semaphore_wait(barrier_sem, 2)

        def partial_half(c, lo):
            rows = x_ref[pl.ds(c * m_blk, m_blk), :]
            return jnp.dot(rows, w_ref[:, lo:lo + nh],
                           preferred_element_type=jnp.float32)

        def copy(src, dst, send_sem, recv_sem, dev):
            return pltpu.make_async_remote_copy(
                src_ref=src, dst_ref=dst, send_sem=send_sem, recv_sem=recv_sem,
                device_id=(dev,), device_id_type=pl.DeviceIdType.MESH,
            )

        bf16 = jnp.bfloat16

        snd_feed[0, :, :] = partial_half((my + 2) % N_DEV, 0).astype(bf16)
        feed_a = copy(snd_feed.at[0], rcv_feed.at[0],
                      feed_send_sems.at[0], feed_sems.at[0], left)
        feed_a.start()
        snd_feed[1, :, :] = partial_half((my + 2) % N_DEV, nh).astype(bf16)
        feed_b = copy(snd_feed.at[1], rcv_feed.at[1],
                      feed_send_sems.at[1], feed_sems.at[1], right)
        feed_b.start()

        snd_leaf[0, :, :] = partial_half((my + 1) % N_DEV, 0).astype(bf16)
        leaf_a = copy(snd_leaf.at[0], rcv_leaf.at[0],
                      leaf_send_sems.at[0], leaf_sems.at[0], right)
        leaf_a.start()
        snd_leaf[1, :, :] = partial_half((my - 1) % N_DEV, nh).astype(bf16)
        leaf_b = copy(snd_leaf.at[1], rcv_leaf.at[1],
                      leaf_send_sems.at[1], leaf_sems.at[1], left)
        leaf_b.start()

        p2a = partial_half((my - 1) % N_DEV, 0)
        p2b = partial_half((my + 1) % N_DEV, nh)
        loc_a = partial_half(my, 0)
        loc_b = partial_half(my, nh)

        feed_a_recv = copy(snd_feed.at[0], rcv_feed.at[0],
                           feed_send_sems.at[0], feed_sems.at[0], right)
        feed_a_recv.wait_recv()
        snd_relay[0, :, :] = (rcv_feed[0, :, :].astype(jnp.float32)
                              + p2a).astype(bf16)
        relay_a = copy(snd_relay.at[0], rcv_red.at[0],
                       red_send_sems.at[0], red_sems.at[0], left)
        relay_a.start()

        feed_b_recv = copy(snd_feed.at[1], rcv_feed.at[1],
                           feed_send_sems.at[1], feed_sems.at[1], left)
        feed_b_recv.wait_recv()
        snd_relay[1, :, :] = (rcv_feed[1, :, :].astype(jnp.float32)
                              + p2b).astype(bf16)
        relay_b = copy(snd_relay.at[1], rcv_red.at[1],
                       red_send_sems.at[1], red_sems.at[1], right)
        relay_b.start()

        leaf_a_recv = copy(snd_leaf.at[0], rcv_leaf.at[0],
                           leaf_send_sems.at[0], leaf_sems.at[0], left)
        red_a_recv = copy(snd_relay.at[0], rcv_red.at[0],
                          red_send_sems.at[0], red_sems.at[0], right)
        leaf_a_recv.wait_recv()
        red_a_recv.wait_recv()
        out_ref[:, :nh] = jnp.maximum(
            loc_a + rcv_leaf[0, :, :].astype(jnp.float32)
            + rcv_red[0, :, :].astype(jnp.float32), 0.0)

        leaf_b_recv = copy(snd_leaf.at[1], rcv_leaf.at[1],
                           leaf_send_sems.at[1], leaf_sems.at[1], right)
        red_b_recv = copy(snd_relay.at[1], rcv_red.at[1],
                          red_send_sems.at[1], red_sems.at[1], left)
        leaf_b_recv.wait_recv()
        red_b_recv.wait_recv()
        out_ref[:, nh:] = jnp.maximum(
            loc_b + rcv_leaf[1, :, :].astype(jnp.float32)
            + rcv_red[1, :, :].astype(jnp.float32), 0.0)

        for d in (feed_a, feed_b, leaf_a, leaf_b, relay_a, relay_b):
            d.wait_send()

    half = (m_blk, nh)
    return pl.pallas_call(
        body,
        out_shape=jax.ShapeDtypeStruct((m_blk, n), jnp.float32),
        in_specs=[
            pl.BlockSpec(memory_space=pltpu.VMEM),
            pl.BlockSpec(memory_space=pltpu.VMEM),
        ],
        out_specs=pl.BlockSpec(memory_space=pltpu.VMEM),
        scratch_shapes=[
            pltpu.VMEM((2,) + half, jnp.bfloat16),
            pltpu.VMEM((2,) + half, jnp.bfloat16),
            pltpu.VMEM((2,) + half, jnp.bfloat16),
            pltpu.VMEM((2,) + half, jnp.bfloat16),
            pltpu.VMEM((2,) + half, jnp.bfloat16),
            pltpu.VMEM((2,) + half, jnp.bfloat16),
            pltpu.SemaphoreType.DMA((2,)),
            pltpu.SemaphoreType.DMA((2,)),
            pltpu.SemaphoreType.DMA((2,)),
            pltpu.SemaphoreType.DMA((2,)),
            pltpu.SemaphoreType.DMA((2,)),
            pltpu.SemaphoreType.DMA((2,)),
        ],
        compiler_params=pltpu.CompilerParams(collective_id=0),
    )(x, w_mat)


# device time: 17656 ns/iter; 1.4780x vs baseline; 1.0031x over previous
import jax
import jax.numpy as jnp
from jax import lax
from jax.experimental import pallas as pl
from jax.experimental.pallas import tpu as pltpu

N_DEV = 4


def kernel(x, w_mat):
    m, _ = x.shape
    _, n = w_mat.shape
    m_blk = m // N_DEV
    nh = n // 2

    def body(x_ref, w_ref, out_ref,
             snd_feed, snd_leaf, snd_relay, rcv_feed, rcv_leaf, rcv_red,
             feed_sems, leaf_sems, red_sems,
             feed_send_sems, leaf_send_sems, red_send_sems):
        my = lax.axis_index("i")
        left = (my - 1) % N_DEV
        right = (my + 1) % N_DEV

        barrier_sem = pltpu.get_barrier_semaphore()
        for nbr in (left, right):
            pl.semaphore_signal(
                barrier_sem, inc=1,
                device_id=(nbr,), device_id_type=pl.DeviceIdType.MESH,
            )
        pl.semaphore_wait(barrier_sem, 2)

        def partial_half(c, lo):
            rows = x_ref[pl.ds(c * m_blk, m_blk), :].astype(jnp.bfloat16)
            cols = w_ref[:, lo:lo + nh].astype(jnp.bfloat16)
            return jnp.dot(rows, cols, preferred_element_type=jnp.float32)

        def copy(src, dst, send_sem, recv_sem, dev):
            return pltpu.make_async_remote_copy(
                src_ref=src, dst_ref=dst, send_sem=send_sem, recv_sem=recv_sem,
                device_id=(dev,), device_id_type=pl.DeviceIdType.MESH,
            )

        bf16 = jnp.bfloat16

        snd_feed[0, :, :] = partial_half((my + 2) % N_DEV, 0).astype(bf16)
        feed_a = copy(snd_feed.at[0], rcv_feed.at[0],
                      feed_send_sems.at[0], feed_sems.at[0], left)
        feed_a.start()
        snd_feed[1, :, :] = partial_half((my + 2) % N_DEV, nh).astype(bf16)
        feed_b = copy(snd_feed.at[1], rcv_feed.at[1],
                      feed_send_sems.at[1], feed_sems.at[1], right)
        feed_b.start()

        snd_leaf[0, :, :] = partial_half((my + 1) % N_DEV, 0).astype(bf16)
        leaf_a = copy(snd_leaf.at[0], rcv_leaf.at[0],
                      leaf_send_sems.at[0], leaf_sems.at[0], right)
        leaf_a.start()
        snd_leaf[1, :, :] = partial_half((my - 1) % N_DEV, nh).astype(bf16)
        leaf_b = copy(snd_leaf.at[1], rcv_leaf.at[1],
                      leaf_send_sems.at[1], leaf_sems.at[1], left)
        leaf_b.start()

        p2a = partial_half((my - 1) % N_DEV, 0)
        p2b = partial_half((my + 1) % N_DEV, nh)
        loc_a = partial_half(my, 0)
        loc_b = partial_half(my, nh)

        feed_a_recv = copy(snd_feed.at[0], rcv_feed.at[0],
                           feed_send_sems.at[0], feed_sems.at[0], right)
        feed_a_recv.wait_recv()
        snd_relay[0, :, :] = (rcv_feed[0, :, :].astype(jnp.float32)
                              + p2a).astype(bf16)
        relay_a = copy(snd_relay.at[0], rcv_red.at[0],
                       red_send_sems.at[0], red_sems.at[0], left)
        relay_a.start()

        feed_b_recv = copy(snd_feed.at[1], rcv_feed.at[1],
                           feed_send_sems.at[1], feed_sems.at[1], left)
        feed_b_recv.wait_recv()
        snd_relay[1, :, :] = (rcv_feed[1, :, :].astype(jnp.float32)
                              + p2b).astype(bf16)
        relay_b = copy(snd_relay.at[1], rcv_red.at[1],
                       red_send_sems.at[1], red_sems.at[1], right)
        relay_b.start()

        leaf_a_recv = copy(snd_leaf.at[0], rcv_leaf.at[0],
                           leaf_send_sems.at[0], leaf_sems.at[0], left)
        red_a_recv = copy(snd_relay.at[0], rcv_red.at[0],
                          red_send_sems.at[0], red_sems.at[0], right)
        leaf_a_recv.wait_recv()
        red_a_recv.wait_recv()
        out_ref[:, :nh] = jnp.maximum(
            loc_a + rcv_leaf[0, :, :].astype(jnp.float32)
            + rcv_red[0, :, :].astype(jnp.float32), 0.0)

        leaf_b_recv = copy(snd_leaf.at[1], rcv_leaf.at[1],
                           leaf_send_sems.at[1], leaf_sems.at[1], right)
        red_b_recv = copy(snd_relay.at[1], rcv_red.at[1],
                          red_send_sems.at[1], red_sems.at[1], left)
        leaf_b_recv.wait_recv()
        red_b_recv.wait_recv()
        out_ref[:, nh:] = jnp.maximum(
            loc_b + rcv_leaf[1, :, :].astype(jnp.float32)
            + rcv_red[1, :, :].astype(jnp.float32), 0.0)

        for d in (feed_a, feed_b, leaf_a, leaf_b, relay_a, relay_b):
            d.wait_send()

    half = (m_blk, nh)
    return pl.pallas_call(
        body,
        out_shape=jax.ShapeDtypeStruct((m_blk, n), jnp.float32),
        in_specs=[
            pl.BlockSpec(memory_space=pltpu.VMEM),
            pl.BlockSpec(memory_space=pltpu.VMEM),
        ],
        out_specs=pl.BlockSpec(memory_space=pltpu.VMEM),
        scratch_shapes=[
            pltpu.VMEM((2,) + half, jnp.bfloat16),
            pltpu.VMEM((2,) + half, jnp.bfloat16),
            pltpu.VMEM((2,) + half, jnp.bfloat16),
            pltpu.VMEM((2,) + half, jnp.bfloat16),
            pltpu.VMEM((2,) + half, jnp.bfloat16),
            pltpu.VMEM((2,) + half, jnp.bfloat16),
            pltpu.SemaphoreType.DMA((2,)),
            pltpu.SemaphoreType.DMA((2,)),
            pltpu.SemaphoreType.DMA((2,)),
            pltpu.SemaphoreType.DMA((2,)),
            pltpu.SemaphoreType.DMA((2,)),
            pltpu.SemaphoreType.DMA((2,)),
        ],
        compiler_params=pltpu.CompilerParams(collective_id=0),
    )(x, w_mat)


# device time: 15357 ns/iter; 1.6992x vs baseline; 1.1497x over previous
import jax
import jax.numpy as jnp
from jax import lax
from jax.experimental import pallas as pl
from jax.experimental.pallas import tpu as pltpu

N_DEV = 4


def kernel(x, w_mat):
    m, _ = x.shape
    _, n = w_mat.shape
    m_blk = m // N_DEV
    nh = n // 2

    def body(x_ref, w_ref, out_ref,
             snd_feed, snd_feed_s, snd_leaf, snd_leaf_s, snd_relay,
             rcv_feed, rcv_feed_s, rcv_leaf, rcv_leaf_s, rcv_red,
             feed_sems, feed_s_sems, leaf_sems, leaf_s_sems, red_sems,
             feed_send, feed_s_send, leaf_send, leaf_s_send, red_send):
        my = lax.axis_index("i")
        left = (my - 1) % N_DEV
        right = (my + 1) % N_DEV

        barrier_sem = pltpu.get_barrier_semaphore()
        for nbr in (left, right):
            pl.semaphore_signal(
                barrier_sem, inc=1,
                device_id=(nbr,), device_id_type=pl.DeviceIdType.MESH,
            )
        pl.semaphore_wait(barrier_sem, 2)

        def partial_half(c, lo):
            rows = x_ref[pl.ds(c * m_blk, m_blk), :]
            return jnp.dot(rows, w_ref[:, lo:lo + nh],
                           preferred_element_type=jnp.float32)

        def copy(src, dst, send_sem, recv_sem, dev):
            return pltpu.make_async_remote_copy(
                src_ref=src, dst_ref=dst, send_sem=send_sem, recv_sem=recv_sem,
                device_id=(dev,), device_id_type=pl.DeviceIdType.MESH,
            )

        bf16 = jnp.bfloat16

        def quant(q):
            amax = jnp.max(jnp.abs(q), axis=0, keepdims=True)
            scale = jnp.maximum(amax, 1e-20) * (1.0 / 127.0)
            qi = jnp.clip(jnp.round(q / scale), -127.0, 127.0).astype(jnp.int8)
            return qi, scale

        def dequant(qbuf, sbuf):
            return qbuf[:, :].astype(jnp.float32) * sbuf[0:1, :]

        def send_quantized(q, data_ref, scale_ref, data_sems, scale_sems,
                           data_rcv, scale_rcv, slot, dev):
            qi, scale = quant(q)
            data_ref[slot, :, :] = qi
            scale_ref[slot, :, :] = scale
            d = copy(data_ref.at[slot], data_rcv.at[slot],
                     data_sems[0].at[slot], data_sems[1].at[slot], dev)
            s = copy(scale_ref.at[slot], scale_rcv.at[slot],
                     scale_sems[0].at[slot], scale_sems[1].at[slot], dev)
            d.start()
            s.start()
            return d, s

        feed_a, feed_a_s = send_quantized(
            partial_half((my + 2) % N_DEV, 0), snd_feed, snd_feed_s,
            (feed_send, feed_sems), (feed_s_send, feed_s_sems),
            rcv_feed, rcv_feed_s, 0, left)
        feed_b, feed_b_s = send_quantized(
            partial_half((my + 2) % N_DEV, nh), snd_feed, snd_feed_s,
            (feed_send, feed_sems), (feed_s_send, feed_s_sems),
            rcv_feed, rcv_feed_s, 1, right)

        leaf_a, leaf_a_s = send_quantized(
            partial_half((my + 1) % N_DEV, 0), snd_leaf, snd_leaf_s,
            (leaf_send, leaf_sems), (leaf_s_send, leaf_s_sems),
            rcv_leaf, rcv_leaf_s, 0, right)
        leaf_b, leaf_b_s = send_quantized(
            partial_half((my - 1) % N_DEV, nh), snd_leaf, snd_leaf_s,
            (leaf_send, leaf_sems), (leaf_s_send, leaf_s_sems),
            rcv_leaf, rcv_leaf_s, 1, left)

        p2a = partial_half((my - 1) % N_DEV, 0)
        p2b = partial_half((my + 1) % N_DEV, nh)
        loc_a = partial_half(my, 0)
        loc_b = partial_half(my, nh)

        def wait_pair(data_ref, scale_ref, data_sems, scale_sems,
                      data_rcv, scale_rcv, slot, dev):
            copy(data_ref.at[slot], data_rcv.at[slot],
                 data_sems[0].at[slot], data_sems[1].at[slot], dev).wait_recv()
            copy(scale_ref.at[slot], scale_rcv.at[slot],
                 scale_sems[0].at[slot], scale_sems[1].at[slot], dev).wait_recv()

        wait_pair(snd_feed, snd_feed_s, (feed_send, feed_sems),
                  (feed_s_send, feed_s_sems), rcv_feed, rcv_feed_s, 0, right)
        snd_relay[0, :, :] = (dequant(rcv_feed[0], rcv_feed_s[0])
                              + p2a).astype(bf16)
        relay_a = copy(snd_relay.at[0], rcv_red.at[0],
                       red_send.at[0], red_sems.at[0], left)
        relay_a.start()

        wait_pair(snd_feed, snd_feed_s, (feed_send, feed_sems),
                  (feed_s_send, feed_s_sems), rcv_feed, rcv_feed_s, 1, left)
        snd_relay[1, :, :] = (dequant(rcv_feed[1], rcv_feed_s[1])
                              + p2b).astype(bf16)
        relay_b = copy(snd_relay.at[1], rcv_red.at[1],
                       red_send.at[1], red_sems.at[1], right)
        relay_b.start()

        wait_pair(snd_leaf, snd_leaf_s, (leaf_send, leaf_sems),
                  (leaf_s_send, leaf_s_sems), rcv_leaf, rcv_leaf_s, 0, left)
        copy(snd_relay.at[0], rcv_red.at[0],
             red_send.at[0], red_sems.at[0], right).wait_recv()
        out_ref[:, :nh] = jnp.maximum(
            loc_a + dequant(rcv_leaf[0], rcv_leaf_s[0])
            + rcv_red[0, :, :].astype(jnp.float32), 0.0)

        wait_pair(snd_leaf, snd_leaf_s, (leaf_send, leaf_sems),
                  (leaf_s_send, leaf_s_sems), rcv_leaf, rcv_leaf_s, 1, right)
        copy(snd_relay.at[1], rcv_red.at[1],
             red_send.at[1], red_sems.at[1], left).wait_recv()
        out_ref[:, nh:] = jnp.maximum(
            loc_b + dequant(rcv_leaf[1], rcv_leaf_s[1])
            + rcv_red[1, :, :].astype(jnp.float32), 0.0)

        for d in (feed_a, feed_a_s, feed_b, feed_b_s,
                  leaf_a, leaf_a_s, leaf_b, leaf_b_s, relay_a, relay_b):
            d.wait_send()

    return pl.pallas_call(
        body,
        out_shape=jax.ShapeDtypeStruct((m_blk, n), jnp.float32),
        in_specs=[
            pl.BlockSpec(memory_space=pltpu.VMEM),
            pl.BlockSpec(memory_space=pltpu.VMEM),
        ],
        out_specs=pl.BlockSpec(memory_space=pltpu.VMEM),
        scratch_shapes=[
            pltpu.VMEM((2, m_blk, nh), jnp.int8),
            pltpu.VMEM((2, 1, nh), jnp.float32),
            pltpu.VMEM((2, m_blk, nh), jnp.int8),
            pltpu.VMEM((2, 1, nh), jnp.float32),
            pltpu.VMEM((2, m_blk, nh), jnp.bfloat16),
            pltpu.VMEM((2, m_blk, nh), jnp.int8),
            pltpu.VMEM((2, 1, nh), jnp.float32),
            pltpu.VMEM((2, m_blk, nh), jnp.int8),
            pltpu.VMEM((2, 1, nh), jnp.float32),
            pltpu.VMEM((2, m_blk, nh), jnp.bfloat16),
            pltpu.SemaphoreType.DMA((2,)),
            pltpu.SemaphoreType.DMA((2,)),
            pltpu.SemaphoreType.DMA((2,)),
            pltpu.SemaphoreType.DMA((2,)),
            pltpu.SemaphoreType.DMA((2,)),
            pltpu.SemaphoreType.DMA((2,)),
            pltpu.SemaphoreType.DMA((2,)),
            pltpu.SemaphoreType.DMA((2,)),
            pltpu.SemaphoreType.DMA((2,)),
            pltpu.SemaphoreType.DMA((2,)),
        ],
        compiler_params=pltpu.CompilerParams(collective_id=0),
    )(x, w_mat)


# device time: 14092 ns/iter; 1.8518x vs baseline; 1.0898x over previous
import jax
import jax.numpy as jnp
from jax import lax
from jax.experimental import pallas as pl
from jax.experimental.pallas import tpu as pltpu

N_DEV = 4


def kernel(x, w_mat):
    m, _ = x.shape
    _, n = w_mat.shape
    m_blk = m // N_DEV
    nh = n // 2

    def body(x_ref, w_ref, out_ref,
             snd_feed, snd_feed_s, snd_leaf, snd_leaf_s, snd_relay, snd_relay_s,
             rcv_feed, rcv_feed_s, rcv_leaf, rcv_leaf_s, rcv_red, rcv_red_s,
             feed_sems, feed_s_sems, leaf_sems, leaf_s_sems, red_sems, red_s_sems,
             feed_send, feed_s_send, leaf_send, leaf_s_send, red_send, red_s_send):
        my = lax.axis_index("i")
        left = (my - 1) % N_DEV
        right = (my + 1) % N_DEV

        barrier_sem = pltpu.get_barrier_semaphore()
        for nbr in (left, right):
            pl.semaphore_signal(
                barrier_sem, inc=1,
                device_id=(nbr,), device_id_type=pl.DeviceIdType.MESH,
            )
        pl.semaphore_wait(barrier_sem, 2)

        def partial_half(c, lo):
            rows = x_ref[pl.ds(c * m_blk, m_blk), :]
            return jnp.dot(rows, w_ref[:, lo:lo + nh],
                           preferred_element_type=jnp.float32)

        def copy(src, dst, send_sem, recv_sem, dev):
            return pltpu.make_async_remote_copy(
                src_ref=src, dst_ref=dst, send_sem=send_sem, recv_sem=recv_sem,
                device_id=(dev,), device_id_type=pl.DeviceIdType.MESH,
            )

        bf16 = jnp.bfloat16

        def quant(q):
            amax = jnp.max(jnp.abs(q), axis=0, keepdims=True)
            scale = jnp.maximum(amax, 1e-20) * (1.0 / 127.0)
            qi = jnp.clip(jnp.round(q / scale), -127.0, 127.0).astype(jnp.int8)
            return qi, scale

        def dequant(qbuf, sbuf):
            return qbuf[:, :].astype(jnp.float32) * sbuf[0:1, :]

        def send_quantized(q, data_ref, scale_ref, data_sems, scale_sems,
                           data_rcv, scale_rcv, slot, dev):
            qi, scale = quant(q)
            data_ref[slot, :, :] = qi
            scale_ref[slot, :, :] = scale
            d = copy(data_ref.at[slot], data_rcv.at[slot],
                     data_sems[0].at[slot], data_sems[1].at[slot], dev)
            s = copy(scale_ref.at[slot], scale_rcv.at[slot],
                     scale_sems[0].at[slot], scale_sems[1].at[slot], dev)
            d.start()
            s.start()
            return d, s

        feed_a, feed_a_s = send_quantized(
            partial_half((my + 2) % N_DEV, 0), snd_feed, snd_feed_s,
            (feed_send, feed_sems), (feed_s_send, feed_s_sems),
            rcv_feed, rcv_feed_s, 0, left)
        feed_b, feed_b_s = send_quantized(
            partial_half((my + 2) % N_DEV, nh), snd_feed, snd_feed_s,
            (feed_send, feed_sems), (feed_s_send, feed_s_sems),
            rcv_feed, rcv_feed_s, 1, right)

        leaf_a, leaf_a_s = send_quantized(
            partial_half((my + 1) % N_DEV, 0), snd_leaf, snd_leaf_s,
            (leaf_send, leaf_sems), (leaf_s_send, leaf_s_sems),
            rcv_leaf, rcv_leaf_s, 0, right)
        leaf_b, leaf_b_s = send_quantized(
            partial_half((my - 1) % N_DEV, nh), snd_leaf, snd_leaf_s,
            (leaf_send, leaf_sems), (leaf_s_send, leaf_s_sems),
            rcv_leaf, rcv_leaf_s, 1, left)

        p2a = partial_half((my - 1) % N_DEV, 0)
        p2b = partial_half((my + 1) % N_DEV, nh)
        loc_a = partial_half(my, 0)
        loc_b = partial_half(my, nh)

        def wait_pair(data_ref, scale_ref, data_sems, scale_sems,
                      data_rcv, scale_rcv, slot, dev):
            copy(data_ref.at[slot], data_rcv.at[slot],
                 data_sems[0].at[slot], data_sems[1].at[slot], dev).wait_recv()
            copy(scale_ref.at[slot], scale_rcv.at[slot],
                 scale_sems[0].at[slot], scale_sems[1].at[slot], dev).wait_recv()

        wait_pair(snd_feed, snd_feed_s, (feed_send, feed_sems),
                  (feed_s_send, feed_s_sems), rcv_feed, rcv_feed_s, 0, right)
        relay_a, relay_a_s = send_quantized(
            dequant(rcv_feed[0], rcv_feed_s[0]) + p2a, snd_relay, snd_relay_s,
            (red_send, red_sems), (red_s_send, red_s_sems),
            rcv_red, rcv_red_s, 0, left)

        wait_pair(snd_feed, snd_feed_s, (feed_send, feed_sems),
                  (feed_s_send, feed_s_sems), rcv_feed, rcv_feed_s, 1, left)
        relay_b, relay_b_s = send_quantized(
            dequant(rcv_feed[1], rcv_feed_s[1]) + p2b, snd_relay, snd_relay_s,
            (red_send, red_sems), (red_s_send, red_s_sems),
            rcv_red, rcv_red_s, 1, right)

        wait_pair(snd_leaf, snd_leaf_s, (leaf_send, leaf_sems),
                  (leaf_s_send, leaf_s_sems), rcv_leaf, rcv_leaf_s, 0, left)
        wait_pair(snd_relay, snd_relay_s, (red_send, red_sems),
                  (red_s_send, red_s_sems), rcv_red, rcv_red_s, 0, right)
        out_ref[:, :nh] = jnp.maximum(
            loc_a + dequant(rcv_leaf[0], rcv_leaf_s[0])
            + dequant(rcv_red[0], rcv_red_s[0]), 0.0)

        wait_pair(snd_leaf, snd_leaf_s, (leaf_send, leaf_sems),
                  (leaf_s_send, leaf_s_sems), rcv_leaf, rcv_leaf_s, 1, right)
        wait_pair(snd_relay, snd_relay_s, (red_send, red_sems),
                  (red_s_send, red_s_sems), rcv_red, rcv_red_s, 1, left)
        out_ref[:, nh:] = jnp.maximum(
            loc_b + dequant(rcv_leaf[1], rcv_leaf_s[1])
            + dequant(rcv_red[1], rcv_red_s[1]), 0.0)

        for d in (feed_a, feed_a_s, feed_b, feed_b_s,
                  leaf_a, leaf_a_s, leaf_b, leaf_b_s,
                  relay_a, relay_a_s, relay_b, relay_b_s):
            d.wait_send()

    return pl.pallas_call(
        body,
        out_shape=jax.ShapeDtypeStruct((m_blk, n), jnp.float32),
        in_specs=[
            pl.BlockSpec(memory_space=pltpu.VMEM),
            pl.BlockSpec(memory_space=pltpu.VMEM),
        ],
        out_specs=pl.BlockSpec(memory_space=pltpu.VMEM),
        scratch_shapes=[
            pltpu.VMEM((2, m_blk, nh), jnp.int8),
            pltpu.VMEM((2, 1, nh), jnp.float32),
            pltpu.VMEM((2, m_blk, nh), jnp.int8),
            pltpu.VMEM((2, 1, nh), jnp.float32),
            pltpu.VMEM((2, m_blk, nh), jnp.int8),
            pltpu.VMEM((2, 1, nh), jnp.float32),
            pltpu.VMEM((2, m_blk, nh), jnp.int8),
            pltpu.VMEM((2, 1, nh), jnp.float32),
            pltpu.VMEM((2, m_blk, nh), jnp.int8),
            pltpu.VMEM((2, 1, nh), jnp.float32),
            pltpu.VMEM((2, m_blk, nh), jnp.int8),
            pltpu.VMEM((2, 1, nh), jnp.float32),
            pltpu.SemaphoreType.DMA((2,)),
            pltpu.SemaphoreType.DMA((2,)),
            pltpu.SemaphoreType.DMA((2,)),
            pltpu.SemaphoreType.DMA((2,)),
            pltpu.SemaphoreType.DMA((2,)),
            pltpu.SemaphoreType.DMA((2,)),
            pltpu.SemaphoreType.DMA((2,)),
            pltpu.SemaphoreType.DMA((2,)),
            pltpu.SemaphoreType.DMA((2,)),
            pltpu.SemaphoreType.DMA((2,)),
            pltpu.SemaphoreType.DMA((2,)),
            pltpu.SemaphoreType.DMA((2,)),
        ],
        compiler_params=pltpu.CompilerParams(collective_id=0),
    )(x, w_mat)


# device time: 14076 ns/iter; 1.8539x vs baseline; 1.0011x over previous
import jax
import jax.numpy as jnp
from jax import lax
from jax.experimental import pallas as pl
from jax.experimental.pallas import tpu as pltpu

N_DEV = 4


def kernel(x, w_mat):
    m, _ = x.shape
    _, n = w_mat.shape
    m_blk = m // N_DEV
    nh = n // 2

    def body(x_ref, w_ref, out_ref,
             snd_feed, snd_feed_s, snd_leaf, snd_leaf_s, snd_relay, snd_relay_s,
             rcv_feed, rcv_feed_s, rcv_leaf, rcv_leaf_s, rcv_red, rcv_red_s,
             feed_sems, feed_s_sems, leaf_sems, leaf_s_sems, red_sems, red_s_sems,
             feed_send, feed_s_send, leaf_send, leaf_s_send, red_send, red_s_send):
        my = lax.axis_index("i")
        left = (my - 1) % N_DEV
        right = (my + 1) % N_DEV

        barrier_sem = pltpu.get_barrier_semaphore()
        for nbr in (left, right):
            pl.semaphore_signal(
                barrier_sem, inc=1,
                device_id=(nbr,), device_id_type=pl.DeviceIdType.MESH,
            )
        pl.semaphore_wait(barrier_sem, 2)

        def partial_half(c, lo):
            rows = x_ref[pl.ds(c * m_blk, m_blk), :]
            return jnp.dot(rows, w_ref[:, lo:lo + nh],
                           preferred_element_type=jnp.float32)

        def copy(src, dst, send_sem, recv_sem, dev):
            return pltpu.make_async_remote_copy(
                src_ref=src, dst_ref=dst, send_sem=send_sem, recv_sem=recv_sem,
                device_id=(dev,), device_id_type=pl.DeviceIdType.MESH,
            )

        def quant(q):
            amax = jnp.max(jnp.abs(q), axis=0, keepdims=True)
            scale = jnp.maximum(amax, 1e-20) * (1.0 / 127.0)
            qi = jnp.clip(jnp.round(q / scale), -127.0, 127.0).astype(jnp.int8)
            return qi, scale

        def dequant(qbuf, sbuf):
            return qbuf[:, :].astype(jnp.float32) * sbuf[0:1, :]

        def send_quantized(q, data_ref, scale_ref, data_sems, scale_sems,
                           data_rcv, scale_rcv, slot, dev):
            qi, scale = quant(q)
            data_ref[slot, :, :] = qi
            scale_ref[slot, :, :] = scale
            d = copy(data_ref.at[slot], data_rcv.at[slot],
                     data_sems[0].at[slot], data_sems[1].at[slot], dev)
            s = copy(scale_ref.at[slot], scale_rcv.at[slot],
                     scale_sems[0].at[slot], scale_sems[1].at[slot], dev)
            d.start()
            s.start()
            return d, s

        feed_a, feed_a_s = send_quantized(
            partial_half((my + 2) % N_DEV, 0), snd_feed, snd_feed_s,
            (feed_send, feed_sems), (feed_s_send, feed_s_sems),
            rcv_feed, rcv_feed_s, 0, left)
        feed_b, feed_b_s = send_quantized(
            partial_half((my + 2) % N_DEV, nh), snd_feed, snd_feed_s,
            (feed_send, feed_sems), (feed_s_send, feed_s_sems),
            rcv_feed, rcv_feed_s, 1, right)

        leaf_a, leaf_a_s = send_quantized(
            partial_half((my + 1) % N_DEV, 0), snd_leaf, snd_leaf_s,
            (leaf_send, leaf_sems), (leaf_s_send, leaf_s_sems),
            rcv_leaf, rcv_leaf_s, 0, right)
        leaf_b, leaf_b_s = send_quantized(
            partial_half((my - 1) % N_DEV, nh), snd_leaf, snd_leaf_s,
            (leaf_send, leaf_sems), (leaf_s_send, leaf_s_sems),
            rcv_leaf, rcv_leaf_s, 1, left)

        p2a = partial_half((my - 1) % N_DEV, 0)
        p2b = partial_half((my + 1) % N_DEV, nh)
        loc_a = partial_half(my, 0)
        loc_b = partial_half(my, nh)

        def wait_pair(data_ref, scale_ref, data_sems, scale_sems,
                      data_rcv, scale_rcv, slot, dev):
            copy(data_ref.at[slot], data_rcv.at[slot],
                 data_sems[0].at[slot], data_sems[1].at[slot], dev).wait_recv()
            copy(scale_ref.at[slot], scale_rcv.at[slot],
                 scale_sems[0].at[slot], scale_sems[1].at[slot], dev).wait_recv()

        wait_pair(snd_feed, snd_feed_s, (feed_send, feed_sems),
                  (feed_s_send, feed_s_sems), rcv_feed, rcv_feed_s, 0, right)
        relay_a, relay_a_s = send_quantized(
            dequant(rcv_feed[0], rcv_feed_s[0]) + p2a, snd_relay, snd_relay_s,
            (red_send, red_sems), (red_s_send, red_s_sems),
            rcv_red, rcv_red_s, 0, left)

        wait_pair(snd_feed, snd_feed_s, (feed_send, feed_sems),
                  (feed_s_send, feed_s_sems), rcv_feed, rcv_feed_s, 1, left)
        relay_b, relay_b_s = send_quantized(
            dequant(rcv_feed[1], rcv_feed_s[1]) + p2b, snd_relay, snd_relay_s,
            (red_send, red_sems), (red_s_send, red_s_sems),
            rcv_red, rcv_red_s, 1, right)

        wait_pair(snd_leaf, snd_leaf_s, (leaf_send, leaf_sems),
                  (leaf_s_send, leaf_s_sems), rcv_leaf, rcv_leaf_s, 0, left)
        wait_pair(snd_relay, snd_relay_s, (red_send, red_sems),
                  (red_s_send, red_s_sems), rcv_red, rcv_red_s, 0, right)
        out_ref[:, :nh] = jnp.maximum(
            loc_a + dequant(rcv_leaf[0], rcv_leaf_s[0])
            + dequant(rcv_red[0], rcv_red_s[0]), 0.0)

        wait_pair(snd_leaf, snd_leaf_s, (leaf_send, leaf_sems),
                  (leaf_s_send, leaf_s_sems), rcv_leaf, rcv_leaf_s, 1, right)
        wait_pair(snd_relay, snd_relay_s, (red_send, red_sems),
                  (red_s_send, red_s_sems), rcv_red, rcv_red_s, 1, left)
        out_ref[:, nh:] = jnp.maximum(
            loc_b + dequant(rcv_leaf[1], rcv_leaf_s[1])
            + dequant(rcv_red[1], rcv_red_s[1]), 0.0)

        for d in (feed_a, feed_a_s, feed_b, feed_b_s,
                  leaf_a, leaf_a_s, leaf_b, leaf_b_s,
                  relay_a, relay_a_s, relay_b, relay_b_s):
            d.wait_send()

    return pl.pallas_call(
        body,
        out_shape=jax.ShapeDtypeStruct((m_blk, n), jnp.float32),
        in_specs=[
            pl.BlockSpec(memory_space=pltpu.VMEM),
            pl.BlockSpec(memory_space=pltpu.VMEM),
        ],
        out_specs=pl.BlockSpec(memory_space=pltpu.VMEM),
        scratch_shapes=[
            pltpu.VMEM((2, m_blk, nh), jnp.int8),
            pltpu.VMEM((2, 1, nh), jnp.float32),
            pltpu.VMEM((2, m_blk, nh), jnp.int8),
            pltpu.VMEM((2, 1, nh), jnp.float32),
            pltpu.VMEM((2, m_blk, nh), jnp.int8),
            pltpu.VMEM((2, 1, nh), jnp.float32),
            pltpu.VMEM((2, m_blk, nh), jnp.int8),
            pltpu.VMEM((2, 1, nh), jnp.float32),
            pltpu.VMEM((2, m_blk, nh), jnp.int8),
            pltpu.VMEM((2, 1, nh), jnp.float32),
            pltpu.VMEM((2, m_blk, nh), jnp.int8),
            pltpu.VMEM((2, 1, nh), jnp.float32),
            pltpu.SemaphoreType.DMA((2,)),
            pltpu.SemaphoreType.DMA((2,)),
            pltpu.SemaphoreType.DMA((2,)),
            pltpu.SemaphoreType.DMA((2,)),
            pltpu.SemaphoreType.DMA((2,)),
            pltpu.SemaphoreType.DMA((2,)),
            pltpu.SemaphoreType.DMA((2,)),
            pltpu.SemaphoreType.DMA((2,)),
            pltpu.SemaphoreType.DMA((2,)),
            pltpu.SemaphoreType.DMA((2,)),
            pltpu.SemaphoreType.DMA((2,)),
            pltpu.SemaphoreType.DMA((2,)),
        ],
        compiler_params=pltpu.CompilerParams(collective_id=0),
    )(x, w_mat)
